# baseline (speedup 1.0000x reference)
import functools

import jax
import jax.numpy as jnp
from jax import lax
from jax.experimental import pallas as pl
from jax.experimental.pallas import tpu as pltpu
from jax.experimental.pallas import tpu_sc as plsc

NUM_LABELS = 100000
D_MODEL = 64
DIM = 128


_ROW_BLK = 2000


def _mlp_body(tab_ref, w1_ref, b1_ref, w2_ref, b2_ref, out_ref):
    i = pl.program_id(0)
    x = tab_ref[...]
    row_ids = lax.broadcasted_iota(jnp.int32, (_ROW_BLK, 1), 0)
    x = jnp.where((row_ids == 0) & (i == 0), 0.0, x)
    h = jnp.dot(x, w1_ref[...], preferred_element_type=jnp.float32) + b1_ref[...]
    h = 0.5 * h * (1.0 + lax.erf(h * 0.7071067811865476))
    out_ref[...] = (
        jnp.dot(h, w2_ref[...], preferred_element_type=jnp.float32) + b2_ref[...]
    )


def _mlp_table(table, W1, b1, W2, b2):
    grid = (NUM_LABELS // _ROW_BLK,)
    return pl.pallas_call(
        _mlp_body,
        grid=grid,
        in_specs=[
            pl.BlockSpec((_ROW_BLK, D_MODEL), lambda i: (i, 0)),
            pl.BlockSpec((D_MODEL, DIM), lambda i: (0, 0)),
            pl.BlockSpec((1, DIM), lambda i: (0, 0)),
            pl.BlockSpec((DIM, DIM), lambda i: (0, 0)),
            pl.BlockSpec((1, DIM), lambda i: (0, 0)),
        ],
        out_specs=pl.BlockSpec((_ROW_BLK, DIM), lambda i: (i, 0)),
        out_shape=jax.ShapeDtypeStruct((NUM_LABELS, DIM), jnp.float32),
    )(table, W1, b1.reshape(1, DIM), W2, b2.reshape(1, DIM))


_NC = 2
_NS = 16
_NW = _NC * _NS
_B_SEQ = 16384
_L_SEQ = 50
_NBUF = 8
_N_PARTS = 4


@functools.cache
def _make_gather(n_seq):
    s_per_w = n_seq // _NW
    n_groups = s_per_w // _NBUF
    mesh = plsc.VectorSubcoreMesh(core_axis_name="c", subcore_axis_name="s")

    @functools.partial(
        pl.kernel,
        mesh=mesh,
        out_type=jax.ShapeDtypeStruct((n_seq, _L_SEQ, DIM), jnp.float32),
        scratch_types=[
            pltpu.VMEM((s_per_w, _L_SEQ), jnp.int32),
            pltpu.VMEM((_NBUF, _L_SEQ, DIM), jnp.float32),
            pltpu.SemaphoreType.DMA((_NBUF,)),
            pltpu.SemaphoreType.DMA((_NBUF,)),
        ],
    )
    def gather_k(tab_hbm, idx_hbm, out_hbm, idx_all, rows, gsem, wsem):
        wid = lax.axis_index("s") * _NC + lax.axis_index("c")
        base = wid * s_per_w
        pltpu.sync_copy(idx_hbm.at[pl.ds(base, s_per_w)], idx_all)

        def gcopy(b, i):
            src = tab_hbm.at[idx_all.at[i]]
            return pltpu.make_async_copy(src, rows.at[b], gsem.at[b])

        def wcopy(b, i):
            dst = out_hbm.at[base + i]
            return pltpu.make_async_copy(rows.at[b], dst, wsem.at[b])

        for b in range(4):
            gcopy(b, b).start()

        def group(k, carry):
            c0 = k * _NBUF
            for b in range(4):
                gcopy(b, c0 + b).wait()
            for b in range(4):
                wcopy(b, c0 + b).start()
            @pl.when(k > 0)
            def _():
                for b in range(4):
                    wcopy(4 + b, c0 - 4 + b).wait()
            for b in range(4):
                gcopy(4 + b, c0 + 4 + b).start()
            for b in range(4):
                gcopy(4 + b, c0 + 4 + b).wait()
            for b in range(4):
                wcopy(4 + b, c0 + 4 + b).start()
            for b in range(4):
                wcopy(b, c0 + b).wait()
            @pl.when(k + 1 < n_groups)
            def _():
                for b in range(4):
                    gcopy(b, c0 + _NBUF + b).start()
            return carry

        lax.fori_loop(0, n_groups, group, 0)
        for b in range(4):
            wcopy(4 + b, (n_groups - 1) * _NBUF + 4 + b).wait()

    return gather_k


def kernel(t, table, W1, b1, W2, b2):
    mlp_tab = _mlp_table(table, W1, b1, W2, b2)
    n_seq = _B_SEQ // _N_PARTS
    g = _make_gather(n_seq)
    parts = [
        g(mlp_tab, lax.slice_in_dim(t, p * n_seq, (p + 1) * n_seq, axis=0))
        for p in range(_N_PARTS)
    ]
    return jnp.concatenate(parts, axis=0)


# speedup vs baseline: 1.6753x; 1.6753x over previous
import functools

import jax
import jax.numpy as jnp
from jax import lax
from jax.experimental import pallas as pl
from jax.experimental.pallas import tpu as pltpu
from jax.experimental.pallas import tpu_sc as plsc

NUM_LABELS = 100000
D_MODEL = 64
DIM = 128


_ROW_BLK = 2000


def _mlp_body(tab_ref, w1_ref, b1_ref, w2_ref, b2_ref, out_ref):
    i = pl.program_id(0)
    x = tab_ref[...]
    row_ids = lax.broadcasted_iota(jnp.int32, (_ROW_BLK, 1), 0)
    x = jnp.where((row_ids == 0) & (i == 0), 0.0, x)
    h = jnp.dot(x, w1_ref[...], preferred_element_type=jnp.float32) + b1_ref[...]
    h = 0.5 * h * (1.0 + lax.erf(h * 0.7071067811865476))
    out_ref[...] = (
        jnp.dot(h, w2_ref[...], preferred_element_type=jnp.float32) + b2_ref[...]
    )


def _mlp_table(table, W1, b1, W2, b2):
    grid = (NUM_LABELS // _ROW_BLK,)
    return pl.pallas_call(
        _mlp_body,
        grid=grid,
        in_specs=[
            pl.BlockSpec((_ROW_BLK, D_MODEL), lambda i: (i, 0)),
            pl.BlockSpec((D_MODEL, DIM), lambda i: (0, 0)),
            pl.BlockSpec((1, DIM), lambda i: (0, 0)),
            pl.BlockSpec((DIM, DIM), lambda i: (0, 0)),
            pl.BlockSpec((1, DIM), lambda i: (0, 0)),
        ],
        out_specs=pl.BlockSpec((_ROW_BLK, DIM), lambda i: (i, 0)),
        out_shape=jax.ShapeDtypeStruct((NUM_LABELS, DIM), jnp.float32),
    )(table, W1, b1.reshape(1, DIM), W2, b2.reshape(1, DIM))


_NC = 2
_NS = 16
_NW = _NC * _NS
_B_SEQ = 16384
_L_SEQ = 50
_NBUF = 8
_SEQ_CHUNK = 2
_N_PARTS = 1


_N_PASS = 2


@functools.cache
def _make_gather(n_seq):
    s_per_w = n_seq // _NW
    n_chunks = s_per_w // _SEQ_CHUNK
    cpp = n_chunks // _N_PASS
    n_groups = cpp // _NBUF
    pair_rows = _SEQ_CHUNK * _L_SEQ
    mesh = plsc.VectorSubcoreMesh(core_axis_name="c", subcore_axis_name="s")

    @functools.partial(
        pl.kernel,
        mesh=mesh,
        out_type=jax.ShapeDtypeStruct((n_seq, _L_SEQ, DIM), jnp.float32),
        scratch_types=[
            pltpu.VMEM((cpp, pair_rows), jnp.int32),
            pltpu.VMEM((_NBUF, pair_rows, DIM), jnp.float32),
            pltpu.SemaphoreType.DMA((_NBUF,)),
            pltpu.SemaphoreType.DMA((_NBUF,)),
        ],
    )
    def gather_k(tab_hbm, idx_hbm, out_hbm, idx_all, rows, gsem, wsem):
        wid = lax.axis_index("s") * _NC + lax.axis_index("c")
        base = wid * s_per_w

        def run_pass(p):
            seq0 = base + p * cpp * _SEQ_CHUNK

            def gcopy(b, j):
                src = tab_hbm.at[idx_all.at[j]]
                return pltpu.make_async_copy(src, rows.at[b], gsem.at[b])

            def wcopies(b, j):
                return [
                    pltpu.make_async_copy(
                        rows.at[b, pl.ds(jj * _L_SEQ, _L_SEQ)],
                        out_hbm.at[seq0 + j * _SEQ_CHUNK + jj],
                        wsem.at[b],
                    )
                    for jj in range(_SEQ_CHUNK)
                ]

            def wstart(b, j):
                for c in wcopies(b, j):
                    c.start()

            def wwait(b, j):
                for c in wcopies(b, j):
                    c.wait()

            pltpu.sync_copy(idx_hbm.at[pl.ds(wid * n_chunks + p * cpp, cpp)],
                            idx_all)
            for b in range(4):
                gcopy(b, b).start()

            def group(k, carry):
                c0 = k * _NBUF
                for b in range(4):
                    gcopy(b, c0 + b).wait()
                for b in range(4):
                    wstart(b, c0 + b)
                @pl.when(k > 0)
                def _():
                    for b in range(4):
                        wwait(4 + b, c0 - 4 + b)
                for b in range(4):
                    gcopy(4 + b, c0 + 4 + b).start()
                for b in range(4):
                    gcopy(4 + b, c0 + 4 + b).wait()
                for b in range(4):
                    wstart(4 + b, c0 + 4 + b)
                for b in range(4):
                    wwait(b, c0 + b)
                @pl.when(k + 1 < n_groups)
                def _():
                    for b in range(4):
                        gcopy(b, c0 + _NBUF + b).start()
                return carry

            lax.fori_loop(0, n_groups, group, 0)
            for b in range(4):
                wwait(4 + b, (n_groups - 1) * _NBUF + 4 + b)

        for p in range(_N_PASS):
            run_pass(p)

    return gather_k


def kernel(t, table, W1, b1, W2, b2):
    mlp_tab = _mlp_table(table, W1, b1, W2, b2)
    t2 = t.reshape(_B_SEQ // _SEQ_CHUNK, _SEQ_CHUNK * _L_SEQ)
    return _make_gather(_B_SEQ)(mlp_tab, t2)
